# baseline (speedup 1.0000x reference)
import jax
import jax.numpy as jnp
from jax.experimental import pallas as pl
from jax.experimental.pallas import tpu as pltpu

GAIN_TANH = 1.5927812
RSQRT2 = 0.7071067811865476


def _fermi_block(x_ref, t0_ref, t1_ref, wa_ref, wb_ref, w2_ref, w3_ref, b_ref,
                 wp0_ref, bp0_ref, wp1_ref, bp1_ref,
                 ho_ref, o0_ref, o1_ref):
    x = x_ref[...]
    t0 = t0_ref[...]
    t1 = t1_ref[...]
    T = x.shape[0]

    parity = jax.lax.broadcasted_iota(jnp.int32, (T, 1), 0) % 2
    xs = jnp.where(parity == 0, jnp.roll(x, -1, axis=0), jnp.roll(x, 1, axis=0))

    u = jnp.dot(x, wa_ref[...], preferred_element_type=jnp.float32)
    u += jnp.dot(xs, wb_ref[...], preferred_element_type=jnp.float32)
    u += jnp.dot(t0, w2_ref[...], preferred_element_type=jnp.float32).reshape(T, x.shape[1])
    u += jnp.dot(t1, w3_ref[...], preferred_element_type=jnp.float32).reshape(T, x.shape[1])
    u += b_ref[...]
    ho_ref[...] = (x + jnp.tanh(u * RSQRT2) * GAIN_TANH) * RSQRT2

    v0 = jnp.dot(t0, wp0_ref[...], preferred_element_type=jnp.float32) + bp0_ref[...]
    o0_ref[...] = (t0 + jnp.tanh(v0) * GAIN_TANH) * RSQRT2
    v1 = jnp.dot(t1, wp1_ref[...], preferred_element_type=jnp.float32) + bp1_ref[...]
    o1_ref[...] = (t1 + jnp.tanh(v1) * GAIN_TANH) * RSQRT2


def kernel(h_one, h_two_0, h_two_1, spins, W_single, b_single, W_global,
           W_pair0, b_pair0, W_pair1, b_pair1):
    N, d_one = h_one.shape
    d_pair = h_two_0.shape[1]
    K = d_one // d_pair

    Wa = W_single[:d_one] + W_global[:d_one]
    Wb = W_global[d_one:]
    Ws2 = W_single[d_one:d_one + d_pair]
    Ws3 = W_single[d_one + d_pair:]
    b = b_single.reshape(1, d_one)

    def pack_u(Wn):
        Wp = jnp.zeros((d_one, K * d_one), jnp.float32)
        for j in range(K):
            Wp = Wp.at[j * d_pair:(j + 1) * d_pair, j * d_one:(j + 1) * d_one].set(Wn)
        return Wp

    def pack_diag(Wn):
        Wp = jnp.zeros((d_one, d_one), jnp.float32)
        for j in range(K):
            Wp = Wp.at[j * d_pair:(j + 1) * d_pair, j * d_pair:(j + 1) * d_pair].set(Wn)
        return Wp

    W2p = pack_u(Ws2)
    W3p = pack_u(Ws3)
    Wp0p = pack_diag(W_pair0)
    Wp1p = pack_diag(W_pair1)
    bp0 = jnp.tile(b_pair0, K).reshape(1, d_one)
    bp1 = jnp.tile(b_pair1, K).reshape(1, d_one)

    Np = N // K
    t0d = h_two_0.reshape(Np, d_one)
    t1d = h_two_1.reshape(Np, d_one)

    T = 4096
    grid = (N // T,)

    full_spec = lambda a: pl.BlockSpec(a.shape, lambda i: (0, 0))

    ho, o0, o1 = pl.pallas_call(
        _fermi_block,
        grid=grid,
        in_specs=[
            pl.BlockSpec((T, d_one), lambda i: (i, 0)),
            pl.BlockSpec((T // K, d_one), lambda i: (i, 0)),
            pl.BlockSpec((T // K, d_one), lambda i: (i, 0)),
            full_spec(Wa), full_spec(Wb), full_spec(W2p), full_spec(W3p),
            full_spec(b),
            full_spec(Wp0p), full_spec(bp0),
            full_spec(Wp1p), full_spec(bp1),
        ],
        out_specs=[
            pl.BlockSpec((T, d_one), lambda i: (i, 0)),
            pl.BlockSpec((T // K, d_one), lambda i: (i, 0)),
            pl.BlockSpec((T // K, d_one), lambda i: (i, 0)),
        ],
        out_shape=[
            jax.ShapeDtypeStruct((N, d_one), jnp.float32),
            jax.ShapeDtypeStruct((Np, d_one), jnp.float32),
            jax.ShapeDtypeStruct((Np, d_one), jnp.float32),
        ],
        compiler_params=pltpu.CompilerParams(
            dimension_semantics=("parallel",),
        ),
    )(h_one, t0d, t1d, Wa, Wb, W2p, W3p, b,
      Wp0p, bp0, Wp1p, bp1)

    return (ho, o0.reshape(N, d_pair), o1.reshape(N, d_pair))


# speedup vs baseline: 1.2352x; 1.2352x over previous
import jax
import jax.numpy as jnp
from jax.experimental import pallas as pl
from jax.experimental.pallas import tpu as pltpu

GAIN_TANH = 1.5927812
RSQRT2 = 0.7071067811865476


def _fermi_block(x_ref, t0_ref, t1_ref, wa_ref, wb_ref, w2_ref, w3_ref, b_ref,
                 wp0_ref, bp0_ref, wp1_ref, bp1_ref,
                 ho_ref, o0_ref, o1_ref):
    x = x_ref[...]
    t0 = t0_ref[...]
    t1 = t1_ref[...]

    parity = jax.lax.broadcasted_iota(jnp.int32, (x.shape[0], 1), 0) % 2
    xs = jnp.where(parity == 0, jnp.roll(x, -1, axis=0), jnp.roll(x, 1, axis=0))

    u = jnp.dot(x, wa_ref[...], preferred_element_type=jnp.float32)
    u += jnp.dot(xs, wb_ref[...], preferred_element_type=jnp.float32)
    u += jnp.dot(t0, w2_ref[...], preferred_element_type=jnp.float32)
    u += jnp.dot(t1, w3_ref[...], preferred_element_type=jnp.float32)
    u += b_ref[...]
    ho_ref[...] = (x + jnp.tanh(u * RSQRT2) * GAIN_TANH) * RSQRT2

    v0 = jnp.dot(t0, wp0_ref[...], preferred_element_type=jnp.float32) + bp0_ref[...]
    o0_ref[...] = (t0 + jnp.tanh(v0) * GAIN_TANH) * RSQRT2
    v1 = jnp.dot(t1, wp1_ref[...], preferred_element_type=jnp.float32) + bp1_ref[...]
    o1_ref[...] = (t1 + jnp.tanh(v1) * GAIN_TANH) * RSQRT2


def kernel(h_one, h_two_0, h_two_1, spins, W_single, b_single, W_global,
           W_pair0, b_pair0, W_pair1, b_pair1):
    N, d_one = h_one.shape
    d_pair = h_two_0.shape[1]

    Wa = W_single[:d_one] + W_global[:d_one]
    Wb = W_global[d_one:]
    Ws2 = W_single[d_one:d_one + d_pair]
    Ws3 = W_single[d_one + d_pair:]
    b = b_single.reshape(1, d_one)
    bp0 = b_pair0.reshape(1, d_pair)
    bp1 = b_pair1.reshape(1, d_pair)

    T = 4096
    grid = (N // T,)

    row_spec = lambda w: pl.BlockSpec((T, w), lambda i: (i, 0))
    full_spec = lambda a: pl.BlockSpec(a.shape, lambda i: (0, 0))

    ho, o0, o1 = pl.pallas_call(
        _fermi_block,
        grid=grid,
        in_specs=[
            row_spec(d_one), row_spec(d_pair), row_spec(d_pair),
            full_spec(Wa), full_spec(Wb), full_spec(Ws2), full_spec(Ws3),
            full_spec(b),
            full_spec(W_pair0), full_spec(bp0),
            full_spec(W_pair1), full_spec(bp1),
        ],
        out_specs=[row_spec(d_one), row_spec(d_pair), row_spec(d_pair)],
        out_shape=[
            jax.ShapeDtypeStruct((N, d_one), jnp.float32),
            jax.ShapeDtypeStruct((N, d_pair), jnp.float32),
            jax.ShapeDtypeStruct((N, d_pair), jnp.float32),
        ],
        compiler_params=pltpu.CompilerParams(
            dimension_semantics=("parallel",),
        ),
    )(h_one, h_two_0, h_two_1, Wa, Wb, Ws2, Ws3, b,
      W_pair0, bp0, W_pair1, bp1)

    return (ho, o0, o1)
